# speedup vs baseline: 7.0191x; 7.0191x over previous
import functools

import jax
import jax.numpy as jnp
from jax import lax
from jax.experimental import pallas as pl
from jax.experimental.pallas import tpu as pltpu
from jax.experimental.pallas import tpu_sc as plsc

B, S, V, D, P = 1024, 512, 100000, 128, 512
N = B * S

NC, NS = 2, 16
NW = NC * NS
ROWS_PER_W = N // NW
CHUNK = 128
NCHUNK = ROWS_PER_W // CHUNK


def _sc_gather(ids_flat, glyph_table):
    mesh = plsc.VectorSubcoreMesh(
        core_axis_name="c", subcore_axis_name="s", num_cores=NC,
        num_subcores=NS)

    @functools.partial(
        pl.kernel,
        out_type=jax.ShapeDtypeStruct((N, D), jnp.float32),
        mesh=mesh,
        scratch_types=[
            pltpu.VMEM((CHUNK,), jnp.int32),
            pltpu.VMEM((CHUNK, D), jnp.float32),
            pltpu.SemaphoreType.DMA,
        ],
    )
    def k(ids_hbm, table_hbm, out_hbm, idx_v, rows_v, sem):
        wid = lax.axis_index("s") * NC + lax.axis_index("c")
        base_w = wid * ROWS_PER_W

        def body(c, _):
            base = base_w + c * CHUNK
            pltpu.sync_copy(ids_hbm.at[pl.ds(base, CHUNK)], idx_v)
            pltpu.async_copy(table_hbm.at[idx_v], rows_v, sem).wait()
            pltpu.sync_copy(rows_v, out_hbm.at[pl.ds(base, CHUNK)])
            return 0

        lax.fori_loop(0, NCHUNK, body, 0)

    return k(ids_flat, glyph_table)


LN_ROWS = 8192


def _ln_body(x_ref, pos_ref, g_ref, b_ref, o_ref):
    x = x_ref[...].reshape(LN_ROWS // S, S, D)
    x = x + pos_ref[...][None, :, :]
    mu = jnp.mean(x, axis=-1, keepdims=True)
    xc = x - mu
    var = jnp.mean(xc * xc, axis=-1, keepdims=True)
    y = xc * lax.rsqrt(var + 1e-12)
    y = y * g_ref[...][0][None, None, :] + b_ref[...][0][None, None, :]
    o_ref[...] = y.reshape(LN_ROWS, D)


def _tc_layernorm(gathered, pos_table, ln_gamma, ln_beta):
    grid = (N // LN_ROWS,)
    return pl.pallas_call(
        _ln_body,
        grid=grid,
        in_specs=[
            pl.BlockSpec((LN_ROWS, D), lambda i: (i, 0)),
            pl.BlockSpec((S, D), lambda i: (0, 0)),
            pl.BlockSpec((1, D), lambda i: (0, 0)),
            pl.BlockSpec((1, D), lambda i: (0, 0)),
        ],
        out_specs=pl.BlockSpec((LN_ROWS, D), lambda i: (i, 0)),
        out_shape=jax.ShapeDtypeStruct((N, D), jnp.float32),
    )(gathered, pos_table, ln_gamma.reshape(1, D), ln_beta.reshape(1, D))


@jax.jit
def kernel(input_ids, pos_table, glyph_table, ln_gamma, ln_beta):
    ids_flat = input_ids.reshape(N)
    gathered = _sc_gather(ids_flat, glyph_table)
    out = _tc_layernorm(gathered, pos_table, ln_gamma, ln_beta)
    return out.reshape(B, S, D), jnp.zeros((), dtype=jnp.float32)


# speedup vs baseline: 9.8155x; 1.3984x over previous
import functools

import jax
import jax.numpy as jnp
from jax import lax
from jax.experimental import pallas as pl
from jax.experimental.pallas import tpu as pltpu
from jax.experimental.pallas import tpu_sc as plsc

B, S, V, D, P = 1024, 512, 100000, 128, 512
N = B * S

NC, NS = 2, 16
NW = NC * NS
ROWS_PER_W = N // NW
GCHUNK = 128
CHUNK = 256
NCHUNK = ROWS_PER_W // CHUNK


def _sc_gather(ids_flat, glyph_table):
    mesh = plsc.VectorSubcoreMesh(
        core_axis_name="c", subcore_axis_name="s", num_cores=NC,
        num_subcores=NS)

    @functools.partial(
        pl.kernel,
        out_type=jax.ShapeDtypeStruct((N, D), jnp.float32),
        mesh=mesh,
        scratch_types=[
            pltpu.VMEM((ROWS_PER_W,), jnp.int32),
            pltpu.VMEM((CHUNK, D), jnp.float32),
            pltpu.VMEM((CHUNK, D), jnp.float32),
            pltpu.SemaphoreType.DMA,
            pltpu.SemaphoreType.DMA,
        ],
    )
    def k(ids_hbm, table_hbm, out_hbm, idx_all, rows0, rows1, sg0, sg1):
        wid = lax.axis_index("s") * NC + lax.axis_index("c")
        base_w = wid * ROWS_PER_W
        pltpu.sync_copy(ids_hbm.at[pl.ds(base_w, ROWS_PER_W)], idx_all)

        def gpair(c, rows, sem):
            i0 = c * CHUNK
            pltpu.async_copy(
                table_hbm.at[idx_all.at[pl.ds(i0, GCHUNK)]],
                rows.at[pl.ds(0, GCHUNK)], sem)
            pltpu.async_copy(
                table_hbm.at[idx_all.at[pl.ds(i0 + GCHUNK, GCHUNK)]],
                rows.at[pl.ds(GCHUNK, GCHUNK)], sem)

        def gwait(rows, sem):
            pltpu.make_async_copy(
                table_hbm.at[pl.ds(0, CHUNK)], rows, sem).wait()

        def wback(c, rows):
            pltpu.sync_copy(rows, out_hbm.at[pl.ds(base_w + c * CHUNK, CHUNK)])

        gpair(0, rows0, sg0)

        def body(j, _):
            c0 = j * 2
            gpair(c0 + 1, rows1, sg1)
            gwait(rows0, sg0)
            wback(c0, rows0)
            gpair(c0 + 2, rows0, sg0)
            gwait(rows1, sg1)
            wback(c0 + 1, rows1)
            return 0

        lax.fori_loop(0, NCHUNK // 2 - 1, body, 0)
        gpair(NCHUNK - 1, rows1, sg1)
        gwait(rows0, sg0)
        wback(NCHUNK - 2, rows0)
        gwait(rows1, sg1)
        wback(NCHUNK - 1, rows1)

    return k(ids_flat, glyph_table)


LN_ROWS = 8192


def _ln_body(x_ref, pos_ref, g_ref, b_ref, o_ref):
    x = x_ref[...].reshape(LN_ROWS // S, S, D)
    x = x + pos_ref[...][None, :, :]
    mu = jnp.mean(x, axis=-1, keepdims=True)
    xc = x - mu
    var = jnp.mean(xc * xc, axis=-1, keepdims=True)
    y = xc * lax.rsqrt(var + 1e-12)
    y = y * g_ref[...][0][None, None, :] + b_ref[...][0][None, None, :]
    o_ref[...] = y.reshape(LN_ROWS, D)


def _tc_layernorm(gathered, pos_table, ln_gamma, ln_beta):
    grid = (N // LN_ROWS,)
    return pl.pallas_call(
        _ln_body,
        grid=grid,
        in_specs=[
            pl.BlockSpec((LN_ROWS, D), lambda i: (i, 0)),
            pl.BlockSpec((S, D), lambda i: (0, 0)),
            pl.BlockSpec((1, D), lambda i: (0, 0)),
            pl.BlockSpec((1, D), lambda i: (0, 0)),
        ],
        out_specs=pl.BlockSpec((LN_ROWS, D), lambda i: (i, 0)),
        out_shape=jax.ShapeDtypeStruct((N, D), jnp.float32),
    )(gathered, pos_table, ln_gamma.reshape(1, D), ln_beta.reshape(1, D))


@jax.jit
def kernel(input_ids, pos_table, glyph_table, ln_gamma, ln_beta):
    ids_flat = input_ids.reshape(N)
    gathered = _sc_gather(ids_flat, glyph_table)
    out = _tc_layernorm(gathered, pos_table, ln_gamma, ln_beta)
    return out.reshape(B, S, D), jnp.zeros((), dtype=jnp.float32)
